# Initial kernel scaffold; baseline (speedup 1.0000x reference)
import jax
import jax.numpy as jnp
from jax.experimental import pallas as pl


def kernel(x, W1, W2):
    raise NotImplementedError("write your pallas kernel here")


# speedup vs baseline: 5.3402x; 5.3402x over previous
import functools

import jax
import jax.numpy as jnp
from jax.experimental import pallas as pl
from jax.experimental.pallas import tpu as pltpu

_B, _C, _N, _K = 4, 3, 4096, 32
_R = 128

_NEG = -3.0e38


def _leaky(v):
    return jnp.maximum(v, 0.2 * v)


def _knn_conv_body(xf_ref, xr_ref, w1a_ref, wb_ref, w2_ref, o_ref):
    xf = xf_ref[0]
    xr = xr_ref[0]

    xxf = jnp.sum(xf * xf, axis=0)
    xxr = jnp.sum(xr * xr, axis=0)
    g = jax.lax.dot_general(
        xr, xf, (((0,), (0,)), ((), ())),
        preferred_element_type=jnp.float32)
    d = 2.0 * g - xxr[:, None] - xxf[None, :]

    base = jax.lax.dot_general(
        xr, wb_ref[...], (((0,), (0,)), ((), ())),
        preferred_element_type=jnp.float32)
    w1a = w1a_ref[...]
    w2 = w2_ref[...]

    acc = jnp.full((xr.shape[1], w2.shape[1]), _NEG, dtype=jnp.float32)
    ii = jax.lax.broadcasted_iota(jnp.int32, d.shape, 1)
    for _ in range(_K):
        m = jnp.max(d, axis=1, keepdims=True)
        am = jnp.max(jnp.where(d >= m, ii, -1), axis=1, keepdims=True)
        hit = ii == am
        oh = jnp.where(hit, 1.0, 0.0)
        nb = jax.lax.dot_general(
            oh, xf, (((1,), (1,)), ((), ())),
            preferred_element_type=jnp.float32)
        d = jnp.where(hit, _NEG, d)
        h1 = _leaky(jax.lax.dot_general(
            nb, w1a, (((1,), (0,)), ((), ())),
            preferred_element_type=jnp.float32) + base)
        h2 = _leaky(jax.lax.dot_general(
            h1, w2, (((1,), (0,)), ((), ())),
            preferred_element_type=jnp.float32))
        acc = jnp.maximum(acc, h2)
    o_ref[0] = acc


@jax.jit
def kernel(x, W1, W2):
    b, c, n = x.shape
    xp = jnp.pad(x, ((0, 0), (0, 8 - c), (0, 0)))
    w1a = jnp.pad(W1[:, :c].T, ((0, 8 - c), (0, 0)))
    wb = jnp.pad((W1[:, c:] - W1[:, :c]).T, ((0, 8 - c), (0, 0)))
    w2t = W2.T

    out = pl.pallas_call(
        _knn_conv_body,
        grid=(b, n // _R),
        in_specs=[
            pl.BlockSpec((1, 8, n), lambda i, j: (i, 0, 0)),
            pl.BlockSpec((1, 8, _R), lambda i, j: (i, 0, j)),
            pl.BlockSpec((8, 64), lambda i, j: (0, 0)),
            pl.BlockSpec((8, 64), lambda i, j: (0, 0)),
            pl.BlockSpec((64, 64), lambda i, j: (0, 0)),
        ],
        out_specs=pl.BlockSpec((1, _R, 64), lambda i, j: (i, j, 0)),
        out_shape=jax.ShapeDtypeStruct((b, n, 64), jnp.float32),
    )(xp, xp, w1a, wb, w2t)
    return jnp.swapaxes(out, 1, 2)
